# Initial kernel scaffold; baseline (speedup 1.0000x reference)
import functools

import jax
import jax.numpy as jnp
from jax import lax
from jax.experimental import pallas as pl
from jax.experimental.pallas import tpu as pltpu
from jax.experimental.pallas import tpu_sc as plsc

N = 10000
E = 320000
IN_DIM = 128
HID = 16

NC = 2
NS = 16
NW = NC * NS
BLK = 128
NBLK = 80
EPAD = NW * NBLK * BLK
NPAD = 10240
RPT = NPAD // NS

_mesh = plsc.VectorSubcoreMesh(core_axis_name="c", subcore_axis_name="s")


@functools.partial(
    pl.kernel,
    out_type=jax.ShapeDtypeStruct((NC, NPAD, 1), jnp.float32),
    mesh=_mesh,
    scratch_types=[
        pltpu.VMEM((NBLK, BLK), jnp.int32),
        pltpu.VMEM((BLK,), jnp.float32),
        pltpu.VMEM((BLK,), jnp.float32),
        pltpu.VMEM_SHARED((NPAD,), jnp.float32),
    ],
)
def _sc_degree(dst_hbm, out_hbm, dstv, ones_v, zbuf, acc):
    c = lax.axis_index("c")
    s = lax.axis_index("s")
    wid = s * NC + c

    def fill(i, _):
        zbuf[pl.ds(i * 16, 16)] = jnp.zeros((16,), jnp.float32)
        ones_v[pl.ds(i * 16, 16)] = jnp.ones((16,), jnp.float32)
        return 0

    lax.fori_loop(0, BLK // 16, fill, 0)

    def zero(i, _):
        pltpu.sync_copy(zbuf, acc.at[pl.ds(s * RPT + i * BLK, BLK)])
        return 0

    lax.fori_loop(0, RPT // BLK, zero, 0)
    plsc.subcore_barrier()

    pltpu.sync_copy(dst_hbm.at[wid], dstv)

    def blk(j, _):
        pltpu.sync_copy(ones_v, acc.at[dstv.at[j]], add=True)
        return 0

    lax.fori_loop(0, NBLK, blk, 0)
    plsc.subcore_barrier()

    pltpu.sync_copy(acc.at[pl.ds(s * RPT, RPT)],
                    out_hbm.at[c, pl.ds(s * RPT, RPT), 0])


@functools.partial(
    pl.kernel,
    out_type=jax.ShapeDtypeStruct((NC, NPAD, HID), jnp.float32),
    mesh=_mesh,
    scratch_types=[
        pltpu.VMEM((NBLK, BLK), jnp.int32),
        pltpu.VMEM((NBLK, BLK), jnp.int32),
        pltpu.VMEM((BLK, HID), jnp.float32),
        pltpu.VMEM((BLK, HID), jnp.float32),
        pltpu.VMEM((BLK, HID), jnp.float32),
        pltpu.VMEM_SHARED((NPAD, HID), jnp.float32),
        pltpu.SemaphoreType.DMA,
        pltpu.SemaphoreType.DMA,
        pltpu.SemaphoreType.DMA,
    ],
)
def _sc_aggregate(g_hbm, src_hbm, dst_hbm, out_hbm,
                  srcv, dstv, m0, m1, zbuf, acc, g0, g1sem, ssem):
    c = lax.axis_index("c")
    s = lax.axis_index("s")
    wid = s * NC + c

    def fill(i, _):
        zbuf[i] = jnp.zeros((HID,), jnp.float32)
        return 0

    lax.fori_loop(0, BLK, fill, 0)

    def zero(i, _):
        pltpu.sync_copy(zbuf, acc.at[pl.ds(s * RPT + i * BLK, BLK)])
        return 0

    lax.fori_loop(0, RPT // BLK, zero, 0)
    plsc.subcore_barrier()

    pltpu.sync_copy(src_hbm.at[wid], srcv)
    pltpu.sync_copy(dst_hbm.at[wid], dstv)

    gsems = (g0, g1sem)
    bufs = (m0, m1)
    pltpu.async_copy(g_hbm.at[srcv.at[0]], m0, g0)

    def blk(j, _):
        for p in range(2):
            @pl.when(lax.rem(j, 2) == p)
            def _():
                @pl.when(j + 1 < NBLK)
                def _():
                    pltpu.async_copy(g_hbm.at[srcv.at[j + 1]],
                                     bufs[1 - p], gsems[1 - p])
                pltpu.make_async_copy(g_hbm.at[srcv.at[j]], bufs[p],
                                      gsems[p]).wait()
                pltpu.async_copy(bufs[p], acc.at[dstv.at[j]], ssem,
                                 add=True)
                pltpu.make_async_copy(bufs[p], acc.at[dstv.at[j]],
                                      ssem).wait()
        return 0

    lax.fori_loop(0, NBLK, blk, 0)
    plsc.subcore_barrier()

    pltpu.sync_copy(acc.at[pl.ds(s * RPT, RPT)],
                    out_hbm.at[c, pl.ds(s * RPT, RPT)])


def _tc_matmul_body(x_ref, w_ref, o_ref):
    o_ref[...] = jnp.dot(x_ref[...], w_ref[...],
                         preferred_element_type=jnp.float32)


def _tc_prep_body(h_ref, deg_ref, g_ref, dinv_ref):
    deg = deg_ref[0] + deg_ref[1] + 1.0
    dinv = lax.rsqrt(deg)
    dinv_ref[...] = dinv
    g_ref[...] = h_ref[...] * dinv


def _tc_layer2_body(q_ref, g1_ref, dinv_ref, b1_ref, w2_ref, g2_ref):
    dinv = dinv_ref[...]
    z = (q_ref[0] + q_ref[1] + g1_ref[...]) * dinv + b1_ref[...]
    z = jnp.maximum(z, 0.0)
    h2 = jnp.dot(z, w2_ref[...], preferred_element_type=jnp.float32)
    g2_ref[...] = h2 * dinv


def _tc_final_body(r_ref, g2_ref, dinv_ref, b2_ref, o_ref):
    logits = (r_ref[0] + r_ref[1] + g2_ref[...]) * dinv_ref[...] + b2_ref[...]
    mask = lax.broadcasted_iota(jnp.int32, logits.shape, 1) < 7
    lm = jnp.where(mask, logits, -jnp.inf)
    mx = jnp.max(lm, axis=1, keepdims=True)
    ex = jnp.where(mask, jnp.exp(logits - mx), 0.0)
    lse = jnp.log(jnp.sum(ex, axis=1, keepdims=True))
    o_ref[...] = logits - mx - lse


_ROWS = 1024
_GRID = NPAD // _ROWS

_tc_matmul = pl.pallas_call(
    _tc_matmul_body,
    grid=(_GRID,),
    in_specs=[pl.BlockSpec((_ROWS, IN_DIM), lambda i: (i, 0)),
              pl.BlockSpec((IN_DIM, HID), lambda i: (0, 0))],
    out_specs=pl.BlockSpec((_ROWS, HID), lambda i: (i, 0)),
    out_shape=jax.ShapeDtypeStruct((NPAD, HID), jnp.float32),
)

_tc_prep = pl.pallas_call(
    _tc_prep_body,
    grid=(_GRID,),
    in_specs=[pl.BlockSpec((_ROWS, HID), lambda i: (i, 0)),
              pl.BlockSpec((NC, _ROWS, 1), lambda i: (0, i, 0))],
    out_specs=[pl.BlockSpec((_ROWS, HID), lambda i: (i, 0)),
               pl.BlockSpec((_ROWS, 1), lambda i: (i, 0))],
    out_shape=[jax.ShapeDtypeStruct((NPAD, HID), jnp.float32),
               jax.ShapeDtypeStruct((NPAD, 1), jnp.float32)],
)

_tc_layer2 = pl.pallas_call(
    _tc_layer2_body,
    grid=(_GRID,),
    in_specs=[pl.BlockSpec((NC, _ROWS, HID), lambda i: (0, i, 0)),
              pl.BlockSpec((_ROWS, HID), lambda i: (i, 0)),
              pl.BlockSpec((_ROWS, 1), lambda i: (i, 0)),
              pl.BlockSpec((1, HID), lambda i: (0, 0)),
              pl.BlockSpec((HID, HID), lambda i: (0, 0))],
    out_specs=pl.BlockSpec((_ROWS, HID), lambda i: (i, 0)),
    out_shape=jax.ShapeDtypeStruct((NPAD, HID), jnp.float32),
)

_tc_final = pl.pallas_call(
    _tc_final_body,
    grid=(_GRID,),
    in_specs=[pl.BlockSpec((NC, _ROWS, HID), lambda i: (0, i, 0)),
              pl.BlockSpec((_ROWS, HID), lambda i: (i, 0)),
              pl.BlockSpec((_ROWS, 1), lambda i: (i, 0)),
              pl.BlockSpec((1, HID), lambda i: (0, 0))],
    out_specs=pl.BlockSpec((_ROWS, HID), lambda i: (i, 0)),
    out_shape=jax.ShapeDtypeStruct((NPAD, HID), jnp.float32),
)


@jax.jit
def kernel(x, edge_index, W1, b1, W2, b2):
    ei = edge_index.astype(jnp.int32)
    pad = jnp.full((EPAD - E,), N, jnp.int32)
    src3 = jnp.concatenate([ei[0], pad]).reshape(NW, NBLK, BLK)
    dst3 = jnp.concatenate([ei[1], pad]).reshape(NW, NBLK, BLK)
    xp = jnp.pad(x, ((0, NPAD - N), (0, 0)))
    w2p = jnp.zeros((HID, HID), jnp.float32).at[:, :7].set(W2)
    b1r = b1.reshape(1, HID)
    b2r = jnp.zeros((1, HID), jnp.float32).at[0, :7].set(b2)

    deg_parts = _sc_degree(dst3)
    h1 = _tc_matmul(xp, W1)
    g1, dinv = _tc_prep(h1, deg_parts)
    q = _sc_aggregate(g1, src3, dst3)
    g2 = _tc_layer2(q, g1, dinv, b1r, w2p)
    r = _sc_aggregate(g2, src3, dst3)
    out = _tc_final(r, g2, dinv, b2r)
    return out[:N, :7]


# speedup vs baseline: 37.7903x; 37.7903x over previous
import functools

import jax
import jax.numpy as jnp
from jax import lax
from jax.experimental import pallas as pl
from jax.experimental.pallas import tpu as pltpu
from jax.experimental.pallas import tpu_sc as plsc

N = 10000
E = 320000
IN_DIM = 128
HID = 16

NC = 2
NS = 16
NW = NC * NS
BLK = 128
NBLK = 80
EPAD = NW * NBLK * BLK
NPAD = 10240
RPT = NPAD // NS

@functools.cache
def _make_sc_degree():
  kern = functools.partial(
      pl.kernel,
      out_type=jax.ShapeDtypeStruct((NC, NPAD), jnp.float32),
      mesh=plsc.VectorSubcoreMesh(core_axis_name="c", subcore_axis_name="s"),
      scratch_types=[
          pltpu.VMEM((NBLK, BLK), jnp.int32),
          pltpu.VMEM((BLK,), jnp.float32),
          pltpu.VMEM((BLK,), jnp.float32),
          pltpu.VMEM_SHARED((NPAD,), jnp.float32),
      ],
  )
  return kern(_sc_degree_body)


def _sc_degree_body(dst_hbm, out_hbm, dstv, ones_v, zbuf, acc):
    c = lax.axis_index("c")
    s = lax.axis_index("s")
    wid = s * NC + c

    def fill(i, _):
        zbuf[pl.ds(i * 16, 16)] = jnp.zeros((16,), jnp.float32)
        ones_v[pl.ds(i * 16, 16)] = jnp.ones((16,), jnp.float32)
        return 0

    lax.fori_loop(0, BLK // 16, fill, 0)

    def zero(i, _):
        pltpu.sync_copy(zbuf, acc.at[pl.ds(s * RPT + i * BLK, BLK)])
        return 0

    lax.fori_loop(0, RPT // BLK, zero, 0)
    plsc.subcore_barrier()

    pltpu.sync_copy(dst_hbm.at[wid], dstv)

    def blk(j, _):
        pltpu.sync_copy(ones_v, acc.at[dstv.at[j]], add=True)
        return 0

    lax.fori_loop(0, NBLK, blk, 0)
    plsc.subcore_barrier()

    pltpu.sync_copy(acc.at[pl.ds(s * RPT, RPT)],
                    out_hbm.at[c, pl.ds(s * RPT, RPT)])


@functools.cache
def _make_sc_aggregate():
  kern = functools.partial(
      pl.kernel,
      out_type=jax.ShapeDtypeStruct((NC, NPAD, HID), jnp.float32),
      mesh=plsc.VectorSubcoreMesh(core_axis_name="c", subcore_axis_name="s"),
      scratch_types=[
          pltpu.VMEM((NBLK, BLK), jnp.int32),
          pltpu.VMEM((NBLK, BLK), jnp.int32),
          pltpu.VMEM((BLK, HID), jnp.float32),
          pltpu.VMEM((BLK, HID), jnp.float32),
          pltpu.VMEM((BLK, HID), jnp.float32),
          pltpu.VMEM_SHARED((NPAD, HID), jnp.float32),
          pltpu.SemaphoreType.DMA,
          pltpu.SemaphoreType.DMA,
          pltpu.SemaphoreType.DMA,
      ],
      compiler_params=pltpu.CompilerParams(use_tc_tiling_on_sc=False),
  )
  return kern(_sc_aggregate_body)


def _sc_aggregate_body(g_hbm, src_hbm, dst_hbm, out_hbm,
                  srcv, dstv, m0, m1, zbuf, acc, g0, g1sem, ssem):
    c = lax.axis_index("c")
    s = lax.axis_index("s")
    wid = s * NC + c

    def fill(i, _):
        zbuf[i] = jnp.zeros((HID,), jnp.float32)
        return 0

    lax.fori_loop(0, BLK, fill, 0)

    def zero(i, _):
        pltpu.sync_copy(zbuf, acc.at[pl.ds(s * RPT + i * BLK, BLK)])
        return 0

    lax.fori_loop(0, RPT // BLK, zero, 0)
    plsc.subcore_barrier()

    pltpu.sync_copy(src_hbm.at[wid], srcv)
    pltpu.sync_copy(dst_hbm.at[wid], dstv)

    gsems = (g0, g1sem)
    bufs = (m0, m1)
    pltpu.async_copy(g_hbm.at[srcv.at[0]], m0, g0)

    def blk(j, _):
        for p in range(2):
            @pl.when(lax.rem(j, 2) == p)
            def _():
                @pl.when(j + 1 < NBLK)
                def _():
                    pltpu.async_copy(g_hbm.at[srcv.at[j + 1]],
                                     bufs[1 - p], gsems[1 - p])
                pltpu.make_async_copy(g_hbm.at[srcv.at[j]], bufs[p],
                                      gsems[p]).wait()
                pltpu.async_copy(bufs[p], acc.at[dstv.at[j]], ssem,
                                 add=True)
                pltpu.make_async_copy(bufs[p], acc.at[dstv.at[j]],
                                      ssem).wait()
        return 0

    lax.fori_loop(0, NBLK, blk, 0)
    plsc.subcore_barrier()

    pltpu.sync_copy(acc.at[pl.ds(s * RPT, RPT)],
                    out_hbm.at[c, pl.ds(s * RPT, RPT)])


def _tc_matmul_body(x_ref, w_ref, o_ref):
    o_ref[...] = jnp.dot(x_ref[...], w_ref[...],
                         preferred_element_type=jnp.float32)


def _tc_prep_body(h_ref, deg_ref, g_ref, dinv_ref):
    deg = deg_ref[0] + deg_ref[1] + 1.0
    dinv = lax.rsqrt(deg)
    dinv_ref[...] = dinv
    g_ref[...] = h_ref[...] * dinv


def _tc_layer2_body(q_ref, g1_ref, dinv_ref, b1_ref, w2_ref, g2_ref):
    dinv = dinv_ref[...]
    z = (q_ref[0] + q_ref[1] + g1_ref[...]) * dinv + b1_ref[...]
    z = jnp.maximum(z, 0.0)
    h2 = jnp.dot(z, w2_ref[...], preferred_element_type=jnp.float32)
    g2_ref[...] = h2 * dinv


def _tc_final_body(r_ref, g2_ref, dinv_ref, b2_ref, o_ref):
    logits = (r_ref[0] + r_ref[1] + g2_ref[...]) * dinv_ref[...] + b2_ref[...]
    mask = lax.broadcasted_iota(jnp.int32, logits.shape, 1) < 7
    lm = jnp.where(mask, logits, -jnp.inf)
    mx = jnp.max(lm, axis=1, keepdims=True)
    ex = jnp.where(mask, jnp.exp(logits - mx), 0.0)
    lse = jnp.log(jnp.sum(ex, axis=1, keepdims=True))
    o_ref[...] = logits - mx - lse


_ROWS = 1024
_GRID = NPAD // _ROWS

_tc_matmul = pl.pallas_call(
    _tc_matmul_body,
    grid=(_GRID,),
    in_specs=[pl.BlockSpec((_ROWS, IN_DIM), lambda i: (i, 0)),
              pl.BlockSpec((IN_DIM, HID), lambda i: (0, 0))],
    out_specs=pl.BlockSpec((_ROWS, HID), lambda i: (i, 0)),
    out_shape=jax.ShapeDtypeStruct((NPAD, HID), jnp.float32),
)

_tc_prep = pl.pallas_call(
    _tc_prep_body,
    grid=(_GRID,),
    in_specs=[pl.BlockSpec((_ROWS, HID), lambda i: (i, 0)),
              pl.BlockSpec((NC, _ROWS, 1), lambda i: (0, i, 0))],
    out_specs=[pl.BlockSpec((_ROWS, HID), lambda i: (i, 0)),
               pl.BlockSpec((_ROWS, 1), lambda i: (i, 0))],
    out_shape=[jax.ShapeDtypeStruct((NPAD, HID), jnp.float32),
               jax.ShapeDtypeStruct((NPAD, 1), jnp.float32)],
)

_tc_layer2 = pl.pallas_call(
    _tc_layer2_body,
    grid=(_GRID,),
    in_specs=[pl.BlockSpec((NC, _ROWS, HID), lambda i: (0, i, 0)),
              pl.BlockSpec((_ROWS, HID), lambda i: (i, 0)),
              pl.BlockSpec((_ROWS, 1), lambda i: (i, 0)),
              pl.BlockSpec((1, HID), lambda i: (0, 0)),
              pl.BlockSpec((HID, HID), lambda i: (0, 0))],
    out_specs=pl.BlockSpec((_ROWS, HID), lambda i: (i, 0)),
    out_shape=jax.ShapeDtypeStruct((NPAD, HID), jnp.float32),
)

_tc_final = pl.pallas_call(
    _tc_final_body,
    grid=(_GRID,),
    in_specs=[pl.BlockSpec((NC, _ROWS, HID), lambda i: (0, i, 0)),
              pl.BlockSpec((_ROWS, HID), lambda i: (i, 0)),
              pl.BlockSpec((_ROWS, 1), lambda i: (i, 0)),
              pl.BlockSpec((1, HID), lambda i: (0, 0))],
    out_specs=pl.BlockSpec((_ROWS, HID), lambda i: (i, 0)),
    out_shape=jax.ShapeDtypeStruct((NPAD, HID), jnp.float32),
)


@jax.jit
def kernel(x, edge_index, W1, b1, W2, b2):
    ei = edge_index.astype(jnp.int32)
    pad = jnp.full((EPAD - E,), N, jnp.int32)
    src3 = jnp.concatenate([ei[0], pad]).reshape(NW, NBLK, BLK)
    dst3 = jnp.concatenate([ei[1], pad]).reshape(NW, NBLK, BLK)
    xp = jnp.pad(x, ((0, NPAD - N), (0, 0)))
    w2p = jnp.zeros((HID, HID), jnp.float32).at[:, :7].set(W2)
    b1r = b1.reshape(1, HID)
    b2r = jnp.zeros((1, HID), jnp.float32).at[0, :7].set(b2)

    sc_degree = _make_sc_degree()
    sc_aggregate = _make_sc_aggregate()
    deg_parts = sc_degree(dst3).reshape(NC, NPAD, 1)
    h1 = _tc_matmul(xp, W1)
    g1, dinv = _tc_prep(h1, deg_parts)
    q = sc_aggregate(g1, src3, dst3)
    g2 = _tc_layer2(q, g1, dinv, b1r, w2p)
    r = sc_aggregate(g2, src3, dst3)
    out = _tc_final(r, g2, dinv, b2r)
    return out[:N, :7]


# speedup vs baseline: 40.1974x; 1.0637x over previous
import functools

import jax
import jax.numpy as jnp
from jax import lax
from jax.experimental import pallas as pl
from jax.experimental.pallas import tpu as pltpu
from jax.experimental.pallas import tpu_sc as plsc

N = 10000
E = 320000
IN_DIM = 128
HID = 16
OUT2 = 8

NC = 2
NS = 16
NW = NC * NS
BLK = 128
NBLK = 80
EPAD = NW * NBLK * BLK
NPAD = 10240
RPT = NPAD // NS

@functools.cache
def _make_sc_degree():
  kern = functools.partial(
      pl.kernel,
      out_type=jax.ShapeDtypeStruct((NC, NPAD), jnp.float32),
      mesh=plsc.VectorSubcoreMesh(core_axis_name="c", subcore_axis_name="s"),
      scratch_types=[
          pltpu.VMEM((NBLK, BLK), jnp.int32),
          pltpu.VMEM((BLK,), jnp.float32),
          pltpu.VMEM((BLK,), jnp.float32),
          pltpu.VMEM_SHARED((NPAD,), jnp.float32),
      ],
  )
  return kern(_sc_degree_body)


def _sc_degree_body(dst_hbm, out_hbm, dstv, ones_v, zbuf, acc):
    c = lax.axis_index("c")
    s = lax.axis_index("s")
    wid = s * NC + c

    def fill(i, _):
        zbuf[pl.ds(i * 16, 16)] = jnp.zeros((16,), jnp.float32)
        ones_v[pl.ds(i * 16, 16)] = jnp.ones((16,), jnp.float32)
        return 0

    lax.fori_loop(0, BLK // 16, fill, 0)

    def zero(i, _):
        pltpu.sync_copy(zbuf, acc.at[pl.ds(s * RPT + i * BLK, BLK)])
        return 0

    lax.fori_loop(0, RPT // BLK, zero, 0)
    plsc.subcore_barrier()

    pltpu.sync_copy(dst_hbm.at[wid], dstv)

    def blk(j, _):
        pltpu.sync_copy(ones_v, acc.at[dstv.at[j]], add=True)
        return 0

    lax.fori_loop(0, NBLK, blk, 0)
    plsc.subcore_barrier()

    pltpu.sync_copy(acc.at[pl.ds(s * RPT, RPT)],
                    out_hbm.at[c, pl.ds(s * RPT, RPT)])


NBUF = 4


@functools.cache
def _make_sc_aggregate(width):
  kern = functools.partial(
      pl.kernel,
      out_type=jax.ShapeDtypeStruct((NC, NPAD, width), jnp.float32),
      mesh=plsc.VectorSubcoreMesh(core_axis_name="c", subcore_axis_name="s"),
      scratch_types=(
          [pltpu.VMEM((NBLK, BLK), jnp.int32),
           pltpu.VMEM((NBLK, BLK), jnp.int32)]
          + [pltpu.VMEM((BLK, width), jnp.float32) for _ in range(NBUF)]
          + [pltpu.VMEM_SHARED((NPAD, width), jnp.float32)]
          + [pltpu.SemaphoreType.DMA for _ in range(NBUF)]
      ),
      compiler_params=pltpu.CompilerParams(use_tc_tiling_on_sc=False),
  )
  return kern(functools.partial(_sc_aggregate_body, width))


def _sc_aggregate_body(width, g_hbm, src_hbm, dst_hbm, zeros_hbm, out_hbm,
                       srcv, dstv, *rest):
    bufs = rest[:NBUF]
    acc = rest[NBUF]
    sems = rest[NBUF + 1:]
    c = lax.axis_index("c")
    s = lax.axis_index("s")
    wid = s * NC + c

    pltpu.sync_copy(zeros_hbm, acc.at[pl.ds(s * RPT, RPT)])
    plsc.subcore_barrier()

    pltpu.sync_copy(src_hbm.at[wid], srcv)
    pltpu.sync_copy(dst_hbm.at[wid], dstv)

    def ring(i, _):
        for k in range(NBUF):
            j = i * NBUF + k

            @pl.when(i > 0)
            def _():
                pltpu.make_async_copy(bufs[k], acc.at[dstv.at[j - NBUF]],
                                      sems[k]).wait()
            pltpu.async_copy(g_hbm.at[srcv.at[j]], bufs[k], sems[k])
        for k in range(NBUF):
            j = i * NBUF + k
            pltpu.make_async_copy(g_hbm.at[srcv.at[j]], bufs[k],
                                  sems[k]).wait()
            pltpu.async_copy(bufs[k], acc.at[dstv.at[j]], sems[k], add=True)
        return 0

    lax.fori_loop(0, NBLK // NBUF, ring, 0)
    for k in range(NBUF):
        pltpu.make_async_copy(bufs[k], acc.at[dstv.at[NBLK - NBUF + k]],
                              sems[k]).wait()
    plsc.subcore_barrier()

    pltpu.sync_copy(acc.at[pl.ds(s * RPT, RPT)],
                    out_hbm.at[c, pl.ds(s * RPT, RPT)])


def _tc_matmul_body(x_ref, w_ref, o_ref):
    o_ref[...] = jnp.dot(x_ref[...], w_ref[...],
                         preferred_element_type=jnp.float32)


def _tc_prep_body(h_ref, deg_ref, g_ref, dinv_ref):
    deg = deg_ref[0] + deg_ref[1] + 1.0
    dinv = lax.rsqrt(deg)
    dinv_ref[...] = dinv
    g_ref[...] = h_ref[...] * dinv


def _tc_layer2_body(q_ref, g1_ref, dinv_ref, b1_ref, w2_ref, g2_ref):
    dinv = dinv_ref[...]
    z = (q_ref[0] + q_ref[1] + g1_ref[...]) * dinv + b1_ref[...]
    z = jnp.maximum(z, 0.0)
    h2 = jnp.dot(z, w2_ref[...], preferred_element_type=jnp.float32)
    g2_ref[...] = h2 * dinv


def _tc_final_body(r_ref, g2_ref, dinv_ref, b2_ref, o_ref):
    logits = (r_ref[0] + r_ref[1] + g2_ref[...]) * dinv_ref[...] + b2_ref[...]
    mask = lax.broadcasted_iota(jnp.int32, logits.shape, 1) < 7
    lm = jnp.where(mask, logits, -jnp.inf)
    mx = jnp.max(lm, axis=1, keepdims=True)
    ex = jnp.where(mask, jnp.exp(logits - mx), 0.0)
    lse = jnp.log(jnp.sum(ex, axis=1, keepdims=True))
    o_ref[...] = logits - mx - lse


_ROWS = 1024
_GRID = NPAD // _ROWS

_tc_matmul = pl.pallas_call(
    _tc_matmul_body,
    grid=(_GRID,),
    in_specs=[pl.BlockSpec((_ROWS, IN_DIM), lambda i: (i, 0)),
              pl.BlockSpec((IN_DIM, HID), lambda i: (0, 0))],
    out_specs=pl.BlockSpec((_ROWS, HID), lambda i: (i, 0)),
    out_shape=jax.ShapeDtypeStruct((NPAD, HID), jnp.float32),
)

_tc_prep = pl.pallas_call(
    _tc_prep_body,
    grid=(_GRID,),
    in_specs=[pl.BlockSpec((_ROWS, HID), lambda i: (i, 0)),
              pl.BlockSpec((NC, _ROWS, 1), lambda i: (0, i, 0))],
    out_specs=[pl.BlockSpec((_ROWS, HID), lambda i: (i, 0)),
               pl.BlockSpec((_ROWS, 1), lambda i: (i, 0))],
    out_shape=[jax.ShapeDtypeStruct((NPAD, HID), jnp.float32),
               jax.ShapeDtypeStruct((NPAD, 1), jnp.float32)],
)

_tc_layer2 = pl.pallas_call(
    _tc_layer2_body,
    grid=(_GRID,),
    in_specs=[pl.BlockSpec((NC, _ROWS, HID), lambda i: (0, i, 0)),
              pl.BlockSpec((_ROWS, HID), lambda i: (i, 0)),
              pl.BlockSpec((_ROWS, 1), lambda i: (i, 0)),
              pl.BlockSpec((1, HID), lambda i: (0, 0)),
              pl.BlockSpec((HID, OUT2), lambda i: (0, 0))],
    out_specs=pl.BlockSpec((_ROWS, OUT2), lambda i: (i, 0)),
    out_shape=jax.ShapeDtypeStruct((NPAD, OUT2), jnp.float32),
)

_tc_final = pl.pallas_call(
    _tc_final_body,
    grid=(_GRID,),
    in_specs=[pl.BlockSpec((NC, _ROWS, OUT2), lambda i: (0, i, 0)),
              pl.BlockSpec((_ROWS, OUT2), lambda i: (i, 0)),
              pl.BlockSpec((_ROWS, 1), lambda i: (i, 0)),
              pl.BlockSpec((1, OUT2), lambda i: (0, 0))],
    out_specs=pl.BlockSpec((_ROWS, OUT2), lambda i: (i, 0)),
    out_shape=jax.ShapeDtypeStruct((NPAD, OUT2), jnp.float32),
)


@jax.jit
def kernel(x, edge_index, W1, b1, W2, b2):
    ei = edge_index.astype(jnp.int32)
    pad = jnp.full((EPAD - E,), N, jnp.int32)
    src3 = jnp.concatenate([ei[0], pad]).reshape(NW, NBLK, BLK)
    dst3 = jnp.concatenate([ei[1], pad]).reshape(NW, NBLK, BLK)
    xp = jnp.pad(x, ((0, NPAD - N), (0, 0)))
    w2p = jnp.zeros((HID, OUT2), jnp.float32).at[:, :7].set(W2)
    b1r = b1.reshape(1, HID)
    b2r = jnp.zeros((1, OUT2), jnp.float32).at[0, :7].set(b2)

    sc_degree = _make_sc_degree()
    deg_parts = sc_degree(dst3).reshape(NC, NPAD, 1)
    h1 = _tc_matmul(xp, W1)
    g1, dinv = _tc_prep(h1, deg_parts)
    z16 = jnp.zeros((RPT, HID), jnp.float32)
    z8 = jnp.zeros((RPT, OUT2), jnp.float32)
    q = _make_sc_aggregate(HID)(g1, src3, dst3, z16)
    g2 = _tc_layer2(q, g1, dinv, b1r, w2p)
    r = _make_sc_aggregate(OUT2)(g2, src3, dst3, z8)
    out = _tc_final(r, g2, dinv, b2r)
    return out[:N, :7]
